# baseline (speedup 1.0000x reference)
import functools

import jax
import jax.numpy as jnp
from jax import lax
from jax.experimental import pallas as pl
from jax.experimental.pallas import tpu as pltpu
from jax.experimental.pallas import tpu_sc as plsc

N = 10000
E = 320000
D = 128

NC, NS, L = 2, 16, 16
NW = NC * NS
CHUNK = 128
N_BLOCKS = 79
N_PAD = N_BLOCKS * 128
CHUNKS_PER_TILE = 80
E_PAD = NW * CHUNKS_PER_TILE * CHUNK
ROWS_PER_TILE = N_PAD // NS

_mesh = plsc.VectorSubcoreMesh(core_axis_name="c", subcore_axis_name="s")


def _tile_ids():
    cid = lax.axis_index("c")
    sid = lax.axis_index("s")
    return cid, sid


@functools.partial(
    pl.kernel,
    out_type=jax.ShapeDtypeStruct((NC * N_PAD, D), jnp.float32),
    mesh=_mesh,
    scratch_types=[
        pltpu.VMEM((2, CHUNK), jnp.int32),
        pltpu.VMEM((2, CHUNK), jnp.int32),
        pltpu.VMEM((2, CHUNK, D), jnp.float32),
        pltpu.VMEM_SHARED((N_PAD, D), jnp.float32),
        pltpu.SemaphoreType.DMA,
        pltpu.SemaphoreType.DMA,
    ],
)
def _sc_aggregate(hp_hbm, src_hbm, dst_hbm, out_hbm,
                  isrc_v, idst_v, rows_v, acc_sh, sem_i, sem_g):
    cid, sid = _tile_ids()
    row0 = sid * ROWS_PER_TILE
    rchunks = [(q * CHUNK, min(CHUNK, ROWS_PER_TILE - q * CHUNK))
               for q in range((ROWS_PER_TILE + CHUNK - 1) // CHUNK)]
    tile_base = (cid * NS + sid) * (CHUNKS_PER_TILE * CHUNK)

    def eslice(ref, i):
        return ref.at[pl.ds(tile_base + i * CHUNK, CHUNK)]

    def issue_idx(i, b):
        pltpu.async_copy(eslice(src_hbm, i), isrc_v.at[b], sem_i)
        pltpu.async_copy(eslice(dst_hbm, i), idst_v.at[b], sem_i)

    def wait_idx(i, b):
        pltpu.make_async_copy(eslice(src_hbm, i), isrc_v.at[b], sem_i).wait()
        pltpu.make_async_copy(eslice(dst_hbm, i), idst_v.at[b], sem_i).wait()

    def issue_gather(i, b):
        pltpu.async_copy(hp_hbm.at[isrc_v.at[b]], rows_v.at[b], sem_g)

    def wait_gather(i, b):
        pltpu.make_async_copy(
            hp_hbm.at[isrc_v.at[b]], rows_v.at[b], sem_g).wait()

    def scatter(i, b):
        pltpu.sync_copy(rows_v.at[b], acc_sh.at[idst_v.at[b]], add=True)

    def fill_zero(r, _):
        for j in range(D // 16):
            rows_v[0, r, pl.ds(j * 16, 16)] = jnp.zeros((16,), jnp.float32)
        return _

    lax.fori_loop(0, CHUNK, fill_zero, None)
    for r0, rn in rchunks:
        pltpu.sync_copy(rows_v.at[0, pl.ds(0, rn)],
                        acc_sh.at[pl.ds(row0 + r0, rn)])
    plsc.subcore_barrier()

    pltpu.sync_copy(eslice(src_hbm, 0), isrc_v.at[0])
    pltpu.sync_copy(eslice(dst_hbm, 0), idst_v.at[0])
    issue_gather(0, 0)
    issue_idx(1, 1)

    def pair(g, _):
        for b in range(2):
            i = 2 * g + b
            wait_idx(i + 1, 1 - b)
            issue_gather(i + 1, 1 - b)
            wait_gather(i, b)
            scatter(i, b)
            issue_idx(i + 2, b)
        return _

    lax.fori_loop(0, (CHUNKS_PER_TILE - 2) // 2, pair, None)
    i = CHUNKS_PER_TILE - 2
    wait_idx(i + 1, 1)
    issue_gather(i + 1, 1)
    wait_gather(i, 0)
    scatter(i, 0)
    wait_gather(i + 1, 1)
    scatter(i + 1, 1)
    plsc.subcore_barrier()

    for r0, rn in rchunks:
        pltpu.sync_copy(acc_sh.at[pl.ds(row0 + r0, rn)],
                        rows_v.at[0, pl.ds(0, rn)])
        pltpu.sync_copy(rows_v.at[0, pl.ds(0, rn)],
                        out_hbm.at[pl.ds(cid * N_PAD + row0 + r0, rn)])


def _dinv_block(da, db, g):
    deg = da[:, 0:1] + db[:, 0:1] + 1.0
    rows = g * 128 + lax.broadcasted_iota(jnp.int32, (128, 1), 0)
    return jnp.where(rows < N, lax.rsqrt(deg), 0.0)


def _tc_in_body(x_ref, w_ref, da_ref, db_ref, o_ref):
    g = pl.program_id(0)
    dinv = _dinv_block(da_ref[...], db_ref[...], g)
    o_ref[...] = jnp.dot(x_ref[...], w_ref[...],
                         preferred_element_type=jnp.float32) * dinv


def _tc_mid_body(aa_ref, ab_ref, hp_ref, b_ref, w_ref, da_ref, db_ref, o_ref):
    g = pl.program_id(0)
    dinv = _dinv_block(da_ref[...], db_ref[...], g)
    z = jnp.maximum(
        dinv * (aa_ref[...] + ab_ref[...] + hp_ref[...]) + b_ref[...], 0.0)
    o_ref[...] = jnp.dot(z, w_ref[...],
                         preferred_element_type=jnp.float32) * dinv


def _tc_out_body(aa_ref, ab_ref, hp_ref, b_ref, da_ref, db_ref, o_ref):
    g = pl.program_id(0)
    dinv = _dinv_block(da_ref[...], db_ref[...], g)
    o_ref[...] = jnp.maximum(
        dinv * (aa_ref[...] + ab_ref[...] + hp_ref[...]) + b_ref[...], 0.0)


_row_spec = pl.BlockSpec((128, D), lambda g: (g, 0))
_w_spec = pl.BlockSpec((D, D), lambda g: (0, 0))
_b_spec = pl.BlockSpec((1, D), lambda g: (0, 0))
_da_spec = pl.BlockSpec((128, D), lambda g: (g, 0))
_db_spec = pl.BlockSpec((128, D), lambda g: (N_BLOCKS + g, 0))
_out_struct = jax.ShapeDtypeStruct((N_PAD, D), jnp.float32)


def _tc_in(x_p, w, deg2):
    return pl.pallas_call(
        _tc_in_body, grid=(N_BLOCKS,),
        in_specs=[_row_spec, _w_spec, _da_spec, _db_spec],
        out_specs=_row_spec, out_shape=_out_struct,
    )(x_p, w, deg2, deg2)


def _tc_mid(acc, hp, b, w, deg2):
    return pl.pallas_call(
        _tc_mid_body, grid=(N_BLOCKS,),
        in_specs=[_row_spec,
                  pl.BlockSpec((128, D), lambda g: (N_BLOCKS + g, 0)),
                  _row_spec, _b_spec, _w_spec, _da_spec, _db_spec],
        out_specs=_row_spec, out_shape=_out_struct,
    )(acc, acc, hp, b, w, deg2, deg2)


def _tc_out(acc, hp, b, deg2):
    return pl.pallas_call(
        _tc_out_body, grid=(N_BLOCKS,),
        in_specs=[_row_spec,
                  pl.BlockSpec((128, D), lambda g: (N_BLOCKS + g, 0)),
                  _row_spec, _b_spec, _da_spec, _db_spec],
        out_specs=_row_spec, out_shape=_out_struct,
    )(acc, acc, hp, b, deg2, deg2)


def kernel(x, edge_index, W1, b1, W2, b2):
    pad_e = E_PAD - E
    src_p = jnp.concatenate(
        [edge_index[0], jnp.full((pad_e,), N, jnp.int32)])
    dst_p = jnp.concatenate(
        [edge_index[1], jnp.full((pad_e,), N, jnp.int32)])
    x_p = jnp.concatenate([x, jnp.zeros((N_PAD - N, D), jnp.float32)])
    b1r = b1.reshape(1, D)
    b2r = b2.reshape(1, D)

    deg2 = _sc_aggregate(jnp.ones((N_PAD, D), jnp.float32), dst_p, dst_p)
    h1p = _tc_in(x_p, W1, deg2)
    acc1 = _sc_aggregate(h1p, src_p, dst_p)
    h2p = _tc_mid(acc1, h1p, b1r, W2, deg2)
    acc2 = _sc_aggregate(h2p, src_p, dst_p)
    out = _tc_out(acc2, h2p, b2r, deg2)
    return out[:N]


# speedup vs baseline: 2.9255x; 2.9255x over previous
import functools

import jax
import jax.numpy as jnp
from jax import lax
from jax.experimental import pallas as pl
from jax.experimental.pallas import tpu as pltpu
from jax.experimental.pallas import tpu_sc as plsc

N = 10000
E = 320000
D = 128

NC, NS, L = 2, 16, 16
NW = NC * NS
CHUNK = 128
N_BLOCKS = 79
N_PAD = N_BLOCKS * 128
CHUNKS_PER_TILE = 80
E_PAD = NW * CHUNKS_PER_TILE * CHUNK
ROWS_PER_TILE = N_PAD // NS

_mesh = plsc.VectorSubcoreMesh(core_axis_name="c", subcore_axis_name="s")


def _tile_ids():
    cid = lax.axis_index("c")
    sid = lax.axis_index("s")
    return cid, sid


@functools.partial(
    pl.kernel,
    out_type=jax.ShapeDtypeStruct((NC * N_PAD, D), jnp.float32),
    mesh=_mesh,
    scratch_types=[
        pltpu.VMEM((2, CHUNK), jnp.int32),
        pltpu.VMEM((2, CHUNK), jnp.int32),
        pltpu.VMEM((2, CHUNK, D), jnp.float32),
        pltpu.VMEM_SHARED((N_PAD, D), jnp.float32),
        pltpu.SemaphoreType.DMA,
        pltpu.SemaphoreType.DMA,
    ],
)
def _sc_aggregate(hp_hbm, src_hbm, dst_hbm, out_hbm,
                  isrc_v, idst_v, rows_v, acc_sh, sem_i, sem_g):
    cid, sid = _tile_ids()
    row0 = sid * ROWS_PER_TILE
    rchunks = [(q * CHUNK, min(CHUNK, ROWS_PER_TILE - q * CHUNK))
               for q in range((ROWS_PER_TILE + CHUNK - 1) // CHUNK)]
    tile_base = (cid * NS + sid) * (CHUNKS_PER_TILE * CHUNK)

    def eslice(ref, i):
        return ref.at[pl.ds(tile_base + i * CHUNK, CHUNK)]

    def issue_idx(i, b):
        pltpu.async_copy(eslice(src_hbm, i), isrc_v.at[b], sem_i)
        pltpu.async_copy(eslice(dst_hbm, i), idst_v.at[b], sem_i)

    def wait_idx(i, b):
        pltpu.make_async_copy(eslice(src_hbm, i), isrc_v.at[b], sem_i).wait()
        pltpu.make_async_copy(eslice(dst_hbm, i), idst_v.at[b], sem_i).wait()

    def issue_gather(i, b):
        pltpu.async_copy(hp_hbm.at[isrc_v.at[b]], rows_v.at[b], sem_g)

    def wait_gather(i, b):
        pltpu.make_async_copy(
            hp_hbm.at[isrc_v.at[b]], rows_v.at[b], sem_g).wait()

    def scatter(i, b):
        pltpu.sync_copy(rows_v.at[b], acc_sh.at[idst_v.at[b]], add=True)

    def fill_zero(r, _):
        for j in range(D // 16):
            rows_v[0, r, pl.ds(j * 16, 16)] = jnp.zeros((16,), jnp.float32)
        return _

    lax.fori_loop(0, CHUNK, fill_zero, None)
    for r0, rn in rchunks:
        pltpu.sync_copy(rows_v.at[0, pl.ds(0, rn)],
                        acc_sh.at[pl.ds(row0 + r0, rn)])
    plsc.subcore_barrier()

    pltpu.sync_copy(eslice(src_hbm, 0), isrc_v.at[0])
    pltpu.sync_copy(eslice(dst_hbm, 0), idst_v.at[0])
    issue_gather(0, 0)
    issue_idx(1, 1)

    def pair(g, _):
        for b in range(2):
            i = 2 * g + b
            wait_idx(i + 1, 1 - b)
            issue_gather(i + 1, 1 - b)
            wait_gather(i, b)
            scatter(i, b)
            issue_idx(i + 2, b)
        return _

    lax.fori_loop(0, (CHUNKS_PER_TILE - 2) // 2, pair, None)
    i = CHUNKS_PER_TILE - 2
    wait_idx(i + 1, 1)
    issue_gather(i + 1, 1)
    wait_gather(i, 0)
    scatter(i, 0)
    wait_gather(i + 1, 1)
    scatter(i + 1, 1)
    plsc.subcore_barrier()

    for r0, rn in rchunks:
        pltpu.sync_copy(acc_sh.at[pl.ds(row0 + r0, rn)],
                        rows_v.at[0, pl.ds(0, rn)])
        pltpu.sync_copy(rows_v.at[0, pl.ds(0, rn)],
                        out_hbm.at[pl.ds(cid * N_PAD + row0 + r0, rn)])


def _dinv_block(da, db, g):
    deg = da[:, 0:1] + db[:, 0:1] + 1.0
    rows = g * 128 + lax.broadcasted_iota(jnp.int32, (128, 1), 0)
    return jnp.where(rows < N, lax.rsqrt(deg), 0.0)


def _tc_in_body(x_ref, w_ref, da_ref, db_ref, o_ref):
    g = pl.program_id(0)
    dinv = _dinv_block(da_ref[...], db_ref[...], g)
    o_ref[...] = jnp.dot(x_ref[...], w_ref[...],
                         preferred_element_type=jnp.float32) * dinv


def _tc_mid_body(aa_ref, ab_ref, hp_ref, b_ref, w_ref, da_ref, db_ref, o_ref):
    g = pl.program_id(0)
    dinv = _dinv_block(da_ref[...], db_ref[...], g)
    z = jnp.maximum(
        dinv * (aa_ref[...] + ab_ref[...] + hp_ref[...]) + b_ref[...], 0.0)
    o_ref[...] = jnp.dot(z, w_ref[...],
                         preferred_element_type=jnp.float32) * dinv


def _tc_out_body(aa_ref, ab_ref, hp_ref, b_ref, da_ref, db_ref, o_ref):
    g = pl.program_id(0)
    dinv = _dinv_block(da_ref[...], db_ref[...], g)
    o_ref[...] = jnp.maximum(
        dinv * (aa_ref[...] + ab_ref[...] + hp_ref[...]) + b_ref[...], 0.0)


_row_spec = pl.BlockSpec((128, D), lambda g: (g, 0))
_w_spec = pl.BlockSpec((D, D), lambda g: (0, 0))
_b_spec = pl.BlockSpec((1, D), lambda g: (0, 0))
_da_spec = pl.BlockSpec((128, D), lambda g: (g, 0))
_db_spec = pl.BlockSpec((128, D), lambda g: (N_BLOCKS + g, 0))
_out_struct = jax.ShapeDtypeStruct((N_PAD, D), jnp.float32)


def _tc_in(x_p, w, deg2):
    return pl.pallas_call(
        _tc_in_body, grid=(N_BLOCKS,),
        in_specs=[_row_spec, _w_spec, _da_spec, _db_spec],
        out_specs=_row_spec, out_shape=_out_struct,
    )(x_p, w, deg2, deg2)


def _tc_mid(acc, hp, b, w, deg2):
    return pl.pallas_call(
        _tc_mid_body, grid=(N_BLOCKS,),
        in_specs=[_row_spec,
                  pl.BlockSpec((128, D), lambda g: (N_BLOCKS + g, 0)),
                  _row_spec, _b_spec, _w_spec, _da_spec, _db_spec],
        out_specs=_row_spec, out_shape=_out_struct,
    )(acc, acc, hp, b, w, deg2, deg2)


def _tc_out(acc, hp, b, deg2):
    return pl.pallas_call(
        _tc_out_body, grid=(N_BLOCKS,),
        in_specs=[_row_spec,
                  pl.BlockSpec((128, D), lambda g: (N_BLOCKS + g, 0)),
                  _row_spec, _b_spec, _da_spec, _db_spec],
        out_specs=_row_spec, out_shape=_out_struct,
    )(acc, acc, hp, b, deg2, deg2)


def kernel(x, edge_index, W1, b1, W2, b2):
    pad_e = E_PAD - E
    pad_idx = N + jnp.arange(pad_e, dtype=jnp.int32) % (N_PAD - N)
    src_p = jnp.concatenate([edge_index[0], pad_idx])
    dst_p = jnp.concatenate([edge_index[1], pad_idx])
    x_p = jnp.concatenate([x, jnp.zeros((N_PAD - N, D), jnp.float32)])
    b1r = b1.reshape(1, D)
    b2r = b2.reshape(1, D)

    deg2 = _sc_aggregate(jnp.ones((N_PAD, D), jnp.float32), dst_p, dst_p)
    h1p = _tc_in(x_p, W1, deg2)
    acc1 = _sc_aggregate(h1p, src_p, dst_p)
    h2p = _tc_mid(acc1, h1p, b1r, W2, deg2)
    acc2 = _sc_aggregate(h2p, src_p, dst_p)
    out = _tc_out(acc2, h2p, b2r, deg2)
    return out[:N]


# speedup vs baseline: 3.2608x; 1.1146x over previous
import functools

import jax
import jax.numpy as jnp
from jax import lax
from jax.experimental import pallas as pl
from jax.experimental.pallas import tpu as pltpu
from jax.experimental.pallas import tpu_sc as plsc

N = 10000
E = 320000
D = 128

NC, NS, L = 2, 16, 16
NW = NC * NS
CHUNK = 128
N_BLOCKS = 79
N_PAD = N_BLOCKS * 128
CHUNKS_PER_TILE = 80
E_PAD = NW * CHUNKS_PER_TILE * CHUNK
ROWS_PER_TILE = N_PAD // NS

_mesh = plsc.VectorSubcoreMesh(core_axis_name="c", subcore_axis_name="s")


def _tile_ids():
    cid = lax.axis_index("c")
    sid = lax.axis_index("s")
    return cid, sid


DW = D


@functools.partial(
    pl.kernel,
    out_type=jax.ShapeDtypeStruct((NC * N_PAD, DW), jnp.float32),
    mesh=_mesh,
    scratch_types=[
        pltpu.VMEM((2, CHUNK), jnp.int32),
        pltpu.VMEM((CHUNK, DW), jnp.float32),
        pltpu.VMEM_SHARED((N_PAD, DW), jnp.float32),
        pltpu.SemaphoreType.DMA,
    ],
)
def _sc_degree(dst_hbm, out_hbm, idst_v, ones_v, acc_sh, sem_i):
    cid, sid = _tile_ids()
    row0 = sid * ROWS_PER_TILE
    tile_base = (cid * NS + sid) * (CHUNKS_PER_TILE * CHUNK)
    rchunks = [(q * CHUNK, min(CHUNK, ROWS_PER_TILE - q * CHUNK))
               for q in range((ROWS_PER_TILE + CHUNK - 1) // CHUNK)]

    def eslice(i):
        return dst_hbm.at[pl.ds(tile_base + i * CHUNK, CHUNK)]

    def fill(val):
        def body(r, _):
            for j in range(DW // 16):
                ones_v[r, pl.ds(j * 16, 16)] = jnp.full((16,), val, jnp.float32)
            return _
        lax.fori_loop(0, CHUNK, body, None)

    fill(0.0)
    for r0, rn in rchunks:
        pltpu.sync_copy(ones_v.at[pl.ds(0, rn)],
                        acc_sh.at[pl.ds(row0 + r0, rn)])
    fill(1.0)
    plsc.subcore_barrier()

    pltpu.sync_copy(eslice(0), idst_v.at[0])
    pltpu.async_copy(eslice(1), idst_v.at[1], sem_i)

    def pair(g, _):
        for b in range(2):
            i = 2 * g + b
            pltpu.sync_copy(ones_v, acc_sh.at[idst_v.at[b]], add=True)
            pltpu.make_async_copy(eslice(i + 1), idst_v.at[1 - b], sem_i).wait()
            pltpu.async_copy(eslice(i + 2), idst_v.at[b], sem_i)
        return _

    lax.fori_loop(0, (CHUNKS_PER_TILE - 2) // 2, pair, None)
    i = CHUNKS_PER_TILE - 2
    pltpu.sync_copy(ones_v, acc_sh.at[idst_v.at[0]], add=True)
    pltpu.make_async_copy(eslice(i + 1), idst_v.at[1], sem_i).wait()
    pltpu.sync_copy(ones_v, acc_sh.at[idst_v.at[1]], add=True)
    plsc.subcore_barrier()

    for r0, rn in rchunks:
        pltpu.sync_copy(acc_sh.at[pl.ds(row0 + r0, rn)],
                        ones_v.at[pl.ds(0, rn)])
        pltpu.sync_copy(ones_v.at[pl.ds(0, rn)],
                        out_hbm.at[pl.ds(cid * N_PAD + row0 + r0, rn)])


@functools.partial(
    pl.kernel,
    out_type=jax.ShapeDtypeStruct((NC * N_PAD, D), jnp.float32),
    mesh=_mesh,
    scratch_types=[
        pltpu.VMEM((2, CHUNK), jnp.int32),
        pltpu.VMEM((2, CHUNK), jnp.int32),
        pltpu.VMEM((2, CHUNK, D), jnp.float32),
        pltpu.VMEM_SHARED((N_PAD, D), jnp.float32),
        pltpu.SemaphoreType.DMA,
        pltpu.SemaphoreType.DMA,
    ],
)
def _sc_aggregate(hp_hbm, src_hbm, dst_hbm, out_hbm,
                  isrc_v, idst_v, rows_v, acc_sh, sem_i, sem_g):
    cid, sid = _tile_ids()
    row0 = sid * ROWS_PER_TILE
    rchunks = [(q * CHUNK, min(CHUNK, ROWS_PER_TILE - q * CHUNK))
               for q in range((ROWS_PER_TILE + CHUNK - 1) // CHUNK)]
    tile_base = (cid * NS + sid) * (CHUNKS_PER_TILE * CHUNK)

    def eslice(ref, i):
        return ref.at[pl.ds(tile_base + i * CHUNK, CHUNK)]

    def issue_idx(i, b):
        pltpu.async_copy(eslice(src_hbm, i), isrc_v.at[b], sem_i)
        pltpu.async_copy(eslice(dst_hbm, i), idst_v.at[b], sem_i)

    def wait_idx(i, b):
        pltpu.make_async_copy(eslice(src_hbm, i), isrc_v.at[b], sem_i).wait()
        pltpu.make_async_copy(eslice(dst_hbm, i), idst_v.at[b], sem_i).wait()

    def issue_gather(i, b):
        pltpu.async_copy(hp_hbm.at[isrc_v.at[b]], rows_v.at[b], sem_g)

    def wait_gather(i, b):
        pltpu.make_async_copy(
            hp_hbm.at[isrc_v.at[b]], rows_v.at[b], sem_g).wait()

    def scatter(i, b):
        pltpu.sync_copy(rows_v.at[b], acc_sh.at[idst_v.at[b]], add=True)

    def fill_zero(r, _):
        for j in range(D // 16):
            rows_v[0, r, pl.ds(j * 16, 16)] = jnp.zeros((16,), jnp.float32)
        return _

    lax.fori_loop(0, CHUNK, fill_zero, None)
    for r0, rn in rchunks:
        pltpu.sync_copy(rows_v.at[0, pl.ds(0, rn)],
                        acc_sh.at[pl.ds(row0 + r0, rn)])
    plsc.subcore_barrier()

    pltpu.sync_copy(eslice(src_hbm, 0), isrc_v.at[0])
    pltpu.sync_copy(eslice(dst_hbm, 0), idst_v.at[0])
    issue_gather(0, 0)
    issue_idx(1, 1)

    def pair(g, _):
        for b in range(2):
            i = 2 * g + b
            wait_idx(i + 1, 1 - b)
            issue_gather(i + 1, 1 - b)
            wait_gather(i, b)
            scatter(i, b)
            issue_idx(i + 2, b)
        return _

    lax.fori_loop(0, (CHUNKS_PER_TILE - 2) // 2, pair, None)
    i = CHUNKS_PER_TILE - 2
    wait_idx(i + 1, 1)
    issue_gather(i + 1, 1)
    wait_gather(i, 0)
    scatter(i, 0)
    wait_gather(i + 1, 1)
    scatter(i + 1, 1)
    plsc.subcore_barrier()

    for r0, rn in rchunks:
        pltpu.sync_copy(acc_sh.at[pl.ds(row0 + r0, rn)],
                        rows_v.at[0, pl.ds(0, rn)])
        pltpu.sync_copy(rows_v.at[0, pl.ds(0, rn)],
                        out_hbm.at[pl.ds(cid * N_PAD + row0 + r0, rn)])


def _dinv_block(da, db, g):
    deg = da[:, 0:1] + db[:, 0:1] + 1.0
    rows = g * 128 + lax.broadcasted_iota(jnp.int32, (128, 1), 0)
    return jnp.where(rows < N, lax.rsqrt(deg), 0.0)


def _tc_in_body(x_ref, w_ref, da_ref, db_ref, o_ref):
    g = pl.program_id(0)
    dinv = _dinv_block(da_ref[...], db_ref[...], g)
    o_ref[...] = jnp.dot(x_ref[...], w_ref[...],
                         preferred_element_type=jnp.float32) * dinv


def _tc_mid_body(aa_ref, ab_ref, hp_ref, b_ref, w_ref, da_ref, db_ref, o_ref):
    g = pl.program_id(0)
    dinv = _dinv_block(da_ref[...], db_ref[...], g)
    z = jnp.maximum(
        dinv * (aa_ref[...] + ab_ref[...] + hp_ref[...]) + b_ref[...], 0.0)
    o_ref[...] = jnp.dot(z, w_ref[...],
                         preferred_element_type=jnp.float32) * dinv


def _tc_out_body(aa_ref, ab_ref, hp_ref, b_ref, da_ref, db_ref, o_ref):
    g = pl.program_id(0)
    dinv = _dinv_block(da_ref[...], db_ref[...], g)
    o_ref[...] = jnp.maximum(
        dinv * (aa_ref[...] + ab_ref[...] + hp_ref[...]) + b_ref[...], 0.0)


_row_spec = pl.BlockSpec((128, D), lambda g: (g, 0))
_w_spec = pl.BlockSpec((D, D), lambda g: (0, 0))
_b_spec = pl.BlockSpec((1, D), lambda g: (0, 0))
_da_spec = pl.BlockSpec((128, DW), lambda g: (g, 0))
_db_spec = pl.BlockSpec((128, DW), lambda g: (N_BLOCKS + g, 0))
_out_struct = jax.ShapeDtypeStruct((N_PAD, D), jnp.float32)


def _tc_in(x_p, w, deg2):
    return pl.pallas_call(
        _tc_in_body, grid=(N_BLOCKS,),
        in_specs=[_row_spec, _w_spec, _da_spec, _db_spec],
        out_specs=_row_spec, out_shape=_out_struct,
    )(x_p, w, deg2, deg2)


def _tc_mid(acc, hp, b, w, deg2):
    return pl.pallas_call(
        _tc_mid_body, grid=(N_BLOCKS,),
        in_specs=[_row_spec,
                  pl.BlockSpec((128, D), lambda g: (N_BLOCKS + g, 0)),
                  _row_spec, _b_spec, _w_spec, _da_spec, _db_spec],
        out_specs=_row_spec, out_shape=_out_struct,
    )(acc, acc, hp, b, w, deg2, deg2)


def _tc_out(acc, hp, b, deg2):
    return pl.pallas_call(
        _tc_out_body, grid=(N_BLOCKS,),
        in_specs=[_row_spec,
                  pl.BlockSpec((128, D), lambda g: (N_BLOCKS + g, 0)),
                  _row_spec, _b_spec, _da_spec, _db_spec],
        out_specs=_row_spec, out_shape=_out_struct,
    )(acc, acc, hp, b, deg2, deg2)


def kernel(x, edge_index, W1, b1, W2, b2):
    pad_e = E_PAD - E
    pad_idx = N + jnp.arange(pad_e, dtype=jnp.int32) % (N_PAD - N)
    src_p = jnp.concatenate([edge_index[0], pad_idx])
    dst_p = jnp.concatenate([edge_index[1], pad_idx])
    x_p = jnp.concatenate([x, jnp.zeros((N_PAD - N, D), jnp.float32)])
    b1r = b1.reshape(1, D)
    b2r = b2.reshape(1, D)

    deg2 = _sc_degree(dst_p)
    h1p = _tc_in(x_p, W1, deg2)
    acc1 = _sc_aggregate(h1p, src_p, dst_p)
    h2p = _tc_mid(acc1, h1p, b1r, W2, deg2)
    acc2 = _sc_aggregate(h2p, src_p, dst_p)
    out = _tc_out(acc2, h2p, b2r, deg2)
    return out[:N]


# speedup vs baseline: 3.3025x; 1.0128x over previous
import functools

import jax
import jax.numpy as jnp
from jax import lax
from jax.experimental import pallas as pl
from jax.experimental.pallas import tpu as pltpu
from jax.experimental.pallas import tpu_sc as plsc

N = 10000
E = 320000
D = 128

NC, NS, L = 2, 16, 16
NW = NC * NS
CHUNK = 128
N_BLOCKS = 79
N_PAD = N_BLOCKS * 128
CHUNKS_PER_TILE = 80
E_PAD = NW * CHUNKS_PER_TILE * CHUNK
ROWS_PER_TILE = N_PAD // NS

_mesh = plsc.VectorSubcoreMesh(core_axis_name="c", subcore_axis_name="s")


def _tile_ids():
    cid = lax.axis_index("c")
    sid = lax.axis_index("s")
    return cid, sid


DW = D


@functools.partial(
    pl.kernel,
    out_type=jax.ShapeDtypeStruct((NC * N_PAD, DW), jnp.float32),
    mesh=_mesh,
    scratch_types=[
        pltpu.VMEM((2, CHUNK), jnp.int32),
        pltpu.VMEM((CHUNK, DW), jnp.float32),
        pltpu.VMEM_SHARED((N_PAD, DW), jnp.float32),
        pltpu.SemaphoreType.DMA,
    ],
)
def _sc_degree(edges_hbm, out_hbm, idst_v, ones_v, acc_sh, sem_i):
    cid, sid = _tile_ids()
    row0 = sid * ROWS_PER_TILE
    tile_base = (cid * NS + sid) * (CHUNKS_PER_TILE * CHUNK)
    rchunks = [(q * CHUNK, min(CHUNK, ROWS_PER_TILE - q * CHUNK))
               for q in range((ROWS_PER_TILE + CHUNK - 1) // CHUNK)]

    def eslice(i):
        return edges_hbm.at[pl.ds(E_PAD + tile_base + i * CHUNK, CHUNK)]

    def fill(val):
        def body(r, _):
            for j in range(DW // 16):
                ones_v[r, pl.ds(j * 16, 16)] = jnp.full((16,), val, jnp.float32)
            return _
        lax.fori_loop(0, CHUNK, body, None)

    fill(0.0)
    for r0, rn in rchunks:
        pltpu.sync_copy(ones_v.at[pl.ds(0, rn)],
                        acc_sh.at[pl.ds(row0 + r0, rn)])
    fill(1.0)
    plsc.subcore_barrier()

    pltpu.sync_copy(eslice(0), idst_v.at[0])
    pltpu.async_copy(eslice(1), idst_v.at[1], sem_i)

    def pair(g, _):
        for b in range(2):
            i = 2 * g + b
            pltpu.sync_copy(ones_v, acc_sh.at[idst_v.at[b]], add=True)
            pltpu.make_async_copy(eslice(i + 1), idst_v.at[1 - b], sem_i).wait()
            pltpu.async_copy(eslice(i + 2), idst_v.at[b], sem_i)
        return _

    lax.fori_loop(0, (CHUNKS_PER_TILE - 2) // 2, pair, None)
    i = CHUNKS_PER_TILE - 2
    pltpu.sync_copy(ones_v, acc_sh.at[idst_v.at[0]], add=True)
    pltpu.make_async_copy(eslice(i + 1), idst_v.at[1], sem_i).wait()
    pltpu.sync_copy(ones_v, acc_sh.at[idst_v.at[1]], add=True)
    plsc.subcore_barrier()

    pltpu.sync_copy(acc_sh.at[pl.ds(row0, ROWS_PER_TILE)],
                    out_hbm.at[pl.ds(cid * N_PAD + row0, ROWS_PER_TILE)])


@functools.partial(
    pl.kernel,
    out_type=jax.ShapeDtypeStruct((NC * N_PAD, D), jnp.float32),
    mesh=_mesh,
    scratch_types=[
        pltpu.VMEM((2, CHUNK), jnp.int32),
        pltpu.VMEM((2, CHUNK), jnp.int32),
        pltpu.VMEM((2, CHUNK, D), jnp.float32),
        pltpu.VMEM_SHARED((N_PAD, D), jnp.float32),
        pltpu.SemaphoreType.DMA,
        pltpu.SemaphoreType.DMA,
    ],
)
def _sc_aggregate(hp_hbm, edges_hbm, out_hbm,
                  isrc_v, idst_v, rows_v, acc_sh, sem_i, sem_g):
    cid, sid = _tile_ids()
    row0 = sid * ROWS_PER_TILE
    rchunks = [(q * CHUNK, min(CHUNK, ROWS_PER_TILE - q * CHUNK))
               for q in range((ROWS_PER_TILE + CHUNK - 1) // CHUNK)]
    tile_base = (cid * NS + sid) * (CHUNKS_PER_TILE * CHUNK)

    def eslice(half, i):
        return edges_hbm.at[pl.ds(half * E_PAD + tile_base + i * CHUNK, CHUNK)]

    def issue_idx(i, b):
        pltpu.async_copy(eslice(0, i), isrc_v.at[b], sem_i)
        pltpu.async_copy(eslice(1, i), idst_v.at[b], sem_i)

    def wait_idx(i, b):
        pltpu.make_async_copy(eslice(0, i), isrc_v.at[b], sem_i).wait()
        pltpu.make_async_copy(eslice(1, i), idst_v.at[b], sem_i).wait()

    def issue_gather(i, b):
        pltpu.async_copy(hp_hbm.at[isrc_v.at[b]], rows_v.at[b], sem_g)

    def wait_gather(i, b):
        pltpu.make_async_copy(
            hp_hbm.at[isrc_v.at[b]], rows_v.at[b], sem_g).wait()

    def scatter(i, b):
        pltpu.sync_copy(rows_v.at[b], acc_sh.at[idst_v.at[b]], add=True)

    def fill_zero(r, _):
        for j in range(D // 16):
            rows_v[0, r, pl.ds(j * 16, 16)] = jnp.zeros((16,), jnp.float32)
        return _

    lax.fori_loop(0, CHUNK, fill_zero, None)
    for r0, rn in rchunks:
        pltpu.sync_copy(rows_v.at[0, pl.ds(0, rn)],
                        acc_sh.at[pl.ds(row0 + r0, rn)])
    plsc.subcore_barrier()

    pltpu.sync_copy(eslice(0, 0), isrc_v.at[0])
    pltpu.sync_copy(eslice(1, 0), idst_v.at[0])
    issue_gather(0, 0)
    issue_idx(1, 1)

    def pair(g, _):
        for b in range(2):
            i = 2 * g + b
            wait_idx(i + 1, 1 - b)
            issue_gather(i + 1, 1 - b)
            wait_gather(i, b)
            scatter(i, b)
            issue_idx(i + 2, b)
        return _

    lax.fori_loop(0, (CHUNKS_PER_TILE - 2) // 2, pair, None)
    i = CHUNKS_PER_TILE - 2
    wait_idx(i + 1, 1)
    issue_gather(i + 1, 1)
    wait_gather(i, 0)
    scatter(i, 0)
    wait_gather(i + 1, 1)
    scatter(i + 1, 1)
    plsc.subcore_barrier()

    pltpu.sync_copy(acc_sh.at[pl.ds(row0, ROWS_PER_TILE)],
                    out_hbm.at[pl.ds(cid * N_PAD + row0, ROWS_PER_TILE)])


def _dinv_block(da, db, g):
    deg = da[:, 0:1] + db[:, 0:1] + 1.0
    rows = g * 128 + lax.broadcasted_iota(jnp.int32, (128, 1), 0)
    return jnp.where(rows < N, lax.rsqrt(deg), 0.0)


def _tc_in_body(x_ref, w_ref, da_ref, db_ref, o_ref):
    g = pl.program_id(0)
    dinv = _dinv_block(da_ref[...], db_ref[...], g)
    o_ref[...] = jnp.dot(x_ref[...], w_ref[...],
                         preferred_element_type=jnp.float32) * dinv


def _tc_mid_body(aa_ref, ab_ref, hp_ref, b_ref, w_ref, da_ref, db_ref, o_ref):
    g = pl.program_id(0)
    dinv = _dinv_block(da_ref[...], db_ref[...], g)
    z = jnp.maximum(
        dinv * (aa_ref[...] + ab_ref[...] + hp_ref[...]) + b_ref[...], 0.0)
    o_ref[...] = jnp.dot(z, w_ref[...],
                         preferred_element_type=jnp.float32) * dinv


def _tc_out_body(aa_ref, ab_ref, hp_ref, b_ref, da_ref, db_ref, o_ref):
    g = pl.program_id(0)
    dinv = _dinv_block(da_ref[...], db_ref[...], g)
    o_ref[...] = jnp.maximum(
        dinv * (aa_ref[...] + ab_ref[...] + hp_ref[...]) + b_ref[...], 0.0)


_row_spec = pl.BlockSpec((128, D), lambda g: (g, 0))
_w_spec = pl.BlockSpec((D, D), lambda g: (0, 0))
_b_spec = pl.BlockSpec((1, D), lambda g: (0, 0))
_da_spec = pl.BlockSpec((128, DW), lambda g: (g, 0))
_db_spec = pl.BlockSpec((128, DW), lambda g: (N_BLOCKS + g, 0))
_out_struct = jax.ShapeDtypeStruct((N_PAD, D), jnp.float32)


def _tc_in(x_p, w, deg2):
    return pl.pallas_call(
        _tc_in_body, grid=(N_BLOCKS,),
        in_specs=[_row_spec, _w_spec, _da_spec, _db_spec],
        out_specs=_row_spec, out_shape=_out_struct,
    )(x_p, w, deg2, deg2)


def _tc_mid(acc, hp, b, w, deg2):
    return pl.pallas_call(
        _tc_mid_body, grid=(N_BLOCKS,),
        in_specs=[_row_spec,
                  pl.BlockSpec((128, D), lambda g: (N_BLOCKS + g, 0)),
                  _row_spec, _b_spec, _w_spec, _da_spec, _db_spec],
        out_specs=_row_spec, out_shape=_out_struct,
    )(acc, acc, hp, b, w, deg2, deg2)


def _tc_out(acc, hp, b, deg2):
    return pl.pallas_call(
        _tc_out_body, grid=(N_BLOCKS,),
        in_specs=[_row_spec,
                  pl.BlockSpec((128, D), lambda g: (N_BLOCKS + g, 0)),
                  _row_spec, _b_spec, _da_spec, _db_spec],
        out_specs=_row_spec,
        out_shape=jax.ShapeDtypeStruct((N, D), jnp.float32),
    )(acc, acc, hp, b, deg2, deg2)


def kernel(x, edge_index, W1, b1, W2, b2):
    pad_e = E_PAD - E
    pad_idx = N + jnp.arange(pad_e, dtype=jnp.int32) % (N_PAD - N)
    edges_p = jnp.concatenate(
        [edge_index[0], pad_idx, edge_index[1], pad_idx])
    b1r = b1.reshape(1, D)
    b2r = b2.reshape(1, D)

    deg2 = _sc_degree(edges_p)
    h1p = _tc_in(x, W1, deg2)
    acc1 = _sc_aggregate(h1p, edges_p)
    h2p = _tc_mid(acc1, h1p, b1r, W2, deg2)
    acc2 = _sc_aggregate(h2p, edges_p)
    return _tc_out(acc2, h2p, b2r, deg2)


# speedup vs baseline: 3.5298x; 1.0688x over previous
import functools

import jax
import jax.numpy as jnp
from jax import lax
from jax.experimental import pallas as pl
from jax.experimental.pallas import tpu as pltpu
from jax.experimental.pallas import tpu_sc as plsc

N = 10000
E = 320000
D = 128

NC, NS, L = 2, 16, 16
NW = NC * NS
CHUNK = 128
N_BLOCKS = 79
N_PAD = N_BLOCKS * 128
CHUNKS_PER_TILE = 80
E_PAD = NW * CHUNKS_PER_TILE * CHUNK
ROWS_PER_TILE = N_PAD // NS

_mesh = plsc.VectorSubcoreMesh(core_axis_name="c", subcore_axis_name="s")


def _tile_ids():
    cid = lax.axis_index("c")
    sid = lax.axis_index("s")
    return cid, sid


DW = D


@functools.partial(
    pl.kernel,
    out_type=jax.ShapeDtypeStruct((NC * N_PAD, DW), jnp.float32),
    mesh=_mesh,
    scratch_types=[
        pltpu.VMEM((2, CHUNK), jnp.int32),
        pltpu.VMEM((CHUNK, DW), jnp.float32),
        pltpu.VMEM_SHARED((N_PAD, DW), jnp.float32),
        pltpu.SemaphoreType.DMA,
    ],
)
def _sc_degree(edges_hbm, out_hbm, idst_v, ones_v, acc_sh, sem_i):
    cid, sid = _tile_ids()
    row0 = sid * ROWS_PER_TILE
    tile_base = (cid * NS + sid) * (CHUNKS_PER_TILE * CHUNK)
    rchunks = [(q * CHUNK, min(CHUNK, ROWS_PER_TILE - q * CHUNK))
               for q in range((ROWS_PER_TILE + CHUNK - 1) // CHUNK)]

    def eslice(i):
        return edges_hbm.at[pl.ds(E_PAD + tile_base + i * CHUNK, CHUNK)]

    def fill(val):
        def body(r, _):
            for j in range(DW // 16):
                ones_v[r, pl.ds(j * 16, 16)] = jnp.full((16,), val, jnp.float32)
            return _
        lax.fori_loop(0, CHUNK, body, None)

    fill(0.0)
    for r0, rn in rchunks:
        pltpu.sync_copy(ones_v.at[pl.ds(0, rn)],
                        acc_sh.at[pl.ds(row0 + r0, rn)])
    fill(1.0)
    plsc.subcore_barrier()

    pltpu.sync_copy(eslice(0), idst_v.at[0])
    pltpu.async_copy(eslice(1), idst_v.at[1], sem_i)

    def pair(g, _):
        for b in range(2):
            i = 2 * g + b
            pltpu.sync_copy(ones_v, acc_sh.at[idst_v.at[b]], add=True)
            pltpu.make_async_copy(eslice(i + 1), idst_v.at[1 - b], sem_i).wait()
            pltpu.async_copy(eslice(i + 2), idst_v.at[b], sem_i)
        return _

    lax.fori_loop(0, (CHUNKS_PER_TILE - 2) // 2, pair, None)
    i = CHUNKS_PER_TILE - 2
    pltpu.sync_copy(ones_v, acc_sh.at[idst_v.at[0]], add=True)
    pltpu.make_async_copy(eslice(i + 1), idst_v.at[1], sem_i).wait()
    pltpu.sync_copy(ones_v, acc_sh.at[idst_v.at[1]], add=True)
    plsc.subcore_barrier()

    pltpu.sync_copy(acc_sh.at[pl.ds(row0, ROWS_PER_TILE)],
                    out_hbm.at[pl.ds(cid * N_PAD + row0, ROWS_PER_TILE)])


@functools.partial(
    pl.kernel,
    out_type=jax.ShapeDtypeStruct((NC * N_PAD, D), jnp.float32),
    mesh=_mesh,
    scratch_types=[
        pltpu.VMEM((4, CHUNK), jnp.int32),
        pltpu.VMEM((4, CHUNK), jnp.int32),
        pltpu.VMEM((2, CHUNK, D), jnp.float32),
        pltpu.VMEM_SHARED((N_PAD, D), jnp.float32),
        pltpu.SemaphoreType.DMA,
        pltpu.SemaphoreType.DMA,
        pltpu.SemaphoreType.DMA,
    ],
)
def _sc_aggregate(hp_hbm, edges_hbm, out_hbm,
                  isrc_v, idst_v, rows_v, acc_sh, sem_i, sem_g, sem_s):
    cid, sid = _tile_ids()
    row0 = sid * ROWS_PER_TILE
    rchunks = [(q * CHUNK, min(CHUNK, ROWS_PER_TILE - q * CHUNK))
               for q in range((ROWS_PER_TILE + CHUNK - 1) // CHUNK)]
    tile_base = (cid * NS + sid) * (CHUNKS_PER_TILE * CHUNK)

    def eslice(half, i):
        return edges_hbm.at[pl.ds(half * E_PAD + tile_base + i * CHUNK, CHUNK)]

    def issue_idx(i, b):
        pltpu.async_copy(eslice(0, i), isrc_v.at[b], sem_i)
        pltpu.async_copy(eslice(1, i), idst_v.at[b], sem_i)

    def wait_idx(i, b):
        pltpu.make_async_copy(eslice(0, i), isrc_v.at[b], sem_i).wait()
        pltpu.make_async_copy(eslice(1, i), idst_v.at[b], sem_i).wait()

    def issue_gather(i, s):
        pltpu.async_copy(hp_hbm.at[isrc_v.at[s]], rows_v.at[s % 2], sem_g)

    def wait_gather(i, s):
        pltpu.make_async_copy(
            hp_hbm.at[isrc_v.at[s]], rows_v.at[s % 2], sem_g).wait()

    def issue_scatter(i, s):
        pltpu.async_copy(rows_v.at[s % 2], acc_sh.at[idst_v.at[s]],
                         sem_s, add=True)

    def wait_scatter(i, s):
        pltpu.make_async_copy(rows_v.at[s % 2], acc_sh.at[idst_v.at[s]],
                              sem_s).wait()

    def fill_zero(r, _):
        for j in range(D // 16):
            rows_v[0, r, pl.ds(j * 16, 16)] = jnp.zeros((16,), jnp.float32)
        return _

    lax.fori_loop(0, CHUNK, fill_zero, None)
    for r0, rn in rchunks:
        pltpu.sync_copy(rows_v.at[0, pl.ds(0, rn)],
                        acc_sh.at[pl.ds(row0 + r0, rn)])
    plsc.subcore_barrier()

    def steady(i, s):
        wait_idx(i + 1, (s + 1) % 4)
        wait_scatter(i - 1, (s + 3) % 4)
        issue_gather(i + 1, (s + 1) % 4)
        wait_gather(i, s)
        issue_scatter(i, s)
        issue_idx(i + 2, (s + 2) % 4)

    pltpu.sync_copy(eslice(0, 0), isrc_v.at[0])
    pltpu.sync_copy(eslice(1, 0), idst_v.at[0])
    issue_gather(0, 0)
    issue_idx(1, 1)
    wait_idx(1, 1)
    issue_gather(1, 1)
    wait_gather(0, 0)
    issue_scatter(0, 0)
    issue_idx(2, 2)

    def quad(g, _):
        for b in range(4):
            steady(4 * g + 1 + b, (1 + b) % 4)
        return _

    lax.fori_loop(0, (CHUNKS_PER_TILE - 4) // 4, quad, None)
    n = CHUNKS_PER_TILE
    steady(n - 3, (n - 3) % 4)
    i = n - 2
    wait_idx(i + 1, (i + 1) % 4)
    wait_scatter(i - 1, (i - 1) % 4)
    issue_gather(i + 1, (i + 1) % 4)
    wait_gather(i, i % 4)
    issue_scatter(i, i % 4)
    wait_gather(i + 1, (i + 1) % 4)
    issue_scatter(i + 1, (i + 1) % 4)
    wait_scatter(i, i % 4)
    wait_scatter(i + 1, (i + 1) % 4)
    plsc.subcore_barrier()

    pltpu.sync_copy(acc_sh.at[pl.ds(row0, ROWS_PER_TILE)],
                    out_hbm.at[pl.ds(cid * N_PAD + row0, ROWS_PER_TILE)])


def _dinv_block(da, db, g):
    deg = da[:, 0:1] + db[:, 0:1] + 1.0
    rows = g * 128 + lax.broadcasted_iota(jnp.int32, (128, 1), 0)
    return jnp.where(rows < N, lax.rsqrt(deg), 0.0)


def _tc_in_body(x_ref, w_ref, da_ref, db_ref, o_ref):
    g = pl.program_id(0)
    dinv = _dinv_block(da_ref[...], db_ref[...], g)
    o_ref[...] = jnp.dot(x_ref[...], w_ref[...],
                         preferred_element_type=jnp.float32) * dinv


def _tc_mid_body(aa_ref, ab_ref, hp_ref, b_ref, w_ref, da_ref, db_ref, o_ref):
    g = pl.program_id(0)
    dinv = _dinv_block(da_ref[...], db_ref[...], g)
    z = jnp.maximum(
        dinv * (aa_ref[...] + ab_ref[...] + hp_ref[...]) + b_ref[...], 0.0)
    o_ref[...] = jnp.dot(z, w_ref[...],
                         preferred_element_type=jnp.float32) * dinv


def _tc_out_body(aa_ref, ab_ref, hp_ref, b_ref, da_ref, db_ref, o_ref):
    g = pl.program_id(0)
    dinv = _dinv_block(da_ref[...], db_ref[...], g)
    o_ref[...] = jnp.maximum(
        dinv * (aa_ref[...] + ab_ref[...] + hp_ref[...]) + b_ref[...], 0.0)


_row_spec = pl.BlockSpec((128, D), lambda g: (g, 0))
_w_spec = pl.BlockSpec((D, D), lambda g: (0, 0))
_b_spec = pl.BlockSpec((1, D), lambda g: (0, 0))
_da_spec = pl.BlockSpec((128, DW), lambda g: (g, 0))
_db_spec = pl.BlockSpec((128, DW), lambda g: (N_BLOCKS + g, 0))
_out_struct = jax.ShapeDtypeStruct((N_PAD, D), jnp.float32)


def _tc_in(x_p, w, deg2):
    return pl.pallas_call(
        _tc_in_body, grid=(N_BLOCKS,),
        in_specs=[_row_spec, _w_spec, _da_spec, _db_spec],
        out_specs=_row_spec, out_shape=_out_struct,
    )(x_p, w, deg2, deg2)


def _tc_mid(acc, hp, b, w, deg2):
    return pl.pallas_call(
        _tc_mid_body, grid=(N_BLOCKS,),
        in_specs=[_row_spec,
                  pl.BlockSpec((128, D), lambda g: (N_BLOCKS + g, 0)),
                  _row_spec, _b_spec, _w_spec, _da_spec, _db_spec],
        out_specs=_row_spec, out_shape=_out_struct,
    )(acc, acc, hp, b, w, deg2, deg2)


def _tc_out(acc, hp, b, deg2):
    return pl.pallas_call(
        _tc_out_body, grid=(N_BLOCKS,),
        in_specs=[_row_spec,
                  pl.BlockSpec((128, D), lambda g: (N_BLOCKS + g, 0)),
                  _row_spec, _b_spec, _da_spec, _db_spec],
        out_specs=_row_spec,
        out_shape=jax.ShapeDtypeStruct((N, D), jnp.float32),
    )(acc, acc, hp, b, deg2, deg2)


def kernel(x, edge_index, W1, b1, W2, b2):
    pad_e = E_PAD - E
    pad_idx = N + jnp.arange(pad_e, dtype=jnp.int32) % (N_PAD - N)
    edges_p = jnp.concatenate(
        [edge_index[0], pad_idx, edge_index[1], pad_idx])
    b1r = b1.reshape(1, D)
    b2r = b2.reshape(1, D)

    deg2 = _sc_degree(edges_p)
    h1p = _tc_in(x, W1, deg2)
    acc1 = _sc_aggregate(h1p, edges_p)
    h2p = _tc_mid(acc1, h1p, b1r, W2, deg2)
    acc2 = _sc_aggregate(h2p, edges_p)
    return _tc_out(acc2, h2p, b2r, deg2)


# speedup vs baseline: 3.5375x; 1.0022x over previous
import functools

import jax
import jax.numpy as jnp
from jax import lax
from jax.experimental import pallas as pl
from jax.experimental.pallas import tpu as pltpu
from jax.experimental.pallas import tpu_sc as plsc

N = 10000
E = 320000
D = 128

NC, NS, L = 2, 16, 16
NW = NC * NS
CHUNK = 128
N_BLOCKS = 79
N_PAD = N_BLOCKS * 128
CHUNKS_PER_TILE = 80
E_PAD = NW * CHUNKS_PER_TILE * CHUNK
ROWS_PER_TILE = N_PAD // NS

_mesh = plsc.VectorSubcoreMesh(core_axis_name="c", subcore_axis_name="s")


def _tile_ids():
    cid = lax.axis_index("c")
    sid = lax.axis_index("s")
    return cid, sid


DW = D


@functools.partial(
    pl.kernel,
    out_type=jax.ShapeDtypeStruct((NC * N_PAD, DW), jnp.float32),
    mesh=_mesh,
    scratch_types=[
        pltpu.VMEM((2, CHUNK), jnp.int32),
        pltpu.VMEM((CHUNK, DW), jnp.float32),
        pltpu.VMEM_SHARED((N_PAD, DW), jnp.float32),
        pltpu.SemaphoreType.DMA,
    ],
)
def _sc_degree(edges_hbm, out_hbm, idst_v, ones_v, acc_sh, sem_i):
    cid, sid = _tile_ids()
    row0 = sid * ROWS_PER_TILE
    tile_base = (cid * NS + sid) * (CHUNKS_PER_TILE * CHUNK)
    rchunks = [(q * CHUNK, min(CHUNK, ROWS_PER_TILE - q * CHUNK))
               for q in range((ROWS_PER_TILE + CHUNK - 1) // CHUNK)]

    def eslice(i):
        return edges_hbm.at[pl.ds(E_PAD + tile_base + i * CHUNK, CHUNK)]

    def fill(val):
        def body(r, _):
            for j in range(DW // 16):
                ones_v[r, pl.ds(j * 16, 16)] = jnp.full((16,), val, jnp.float32)
            return _
        lax.fori_loop(0, CHUNK, body, None)

    fill(0.0)
    for r0, rn in rchunks:
        pltpu.sync_copy(ones_v.at[pl.ds(0, rn)],
                        acc_sh.at[pl.ds(row0 + r0, rn)])
    fill(1.0)
    plsc.subcore_barrier()

    pltpu.sync_copy(eslice(0), idst_v.at[0])
    pltpu.async_copy(eslice(1), idst_v.at[1], sem_i)

    def pair(g, _):
        for b in range(2):
            i = 2 * g + b
            pltpu.sync_copy(ones_v, acc_sh.at[idst_v.at[b]], add=True)
            pltpu.make_async_copy(eslice(i + 1), idst_v.at[1 - b], sem_i).wait()
            pltpu.async_copy(eslice(i + 2), idst_v.at[b], sem_i)
        return _

    lax.fori_loop(0, (CHUNKS_PER_TILE - 2) // 2, pair, None)
    i = CHUNKS_PER_TILE - 2
    pltpu.sync_copy(ones_v, acc_sh.at[idst_v.at[0]], add=True)
    pltpu.make_async_copy(eslice(i + 1), idst_v.at[1], sem_i).wait()
    pltpu.sync_copy(ones_v, acc_sh.at[idst_v.at[1]], add=True)
    plsc.subcore_barrier()

    pltpu.sync_copy(acc_sh.at[pl.ds(row0, ROWS_PER_TILE)],
                    out_hbm.at[pl.ds(cid * N_PAD + row0, ROWS_PER_TILE)])


@functools.partial(
    pl.kernel,
    out_type=jax.ShapeDtypeStruct((NC * N_PAD, D), jnp.float32),
    mesh=_mesh,
    scratch_types=[
        pltpu.VMEM((4, CHUNK), jnp.int32),
        pltpu.VMEM((4, CHUNK), jnp.int32),
        pltpu.VMEM((2, CHUNK, D), jnp.float32),
        pltpu.VMEM_SHARED((N_PAD, D), jnp.float32),
        pltpu.SemaphoreType.DMA,
        pltpu.SemaphoreType.DMA,
        pltpu.SemaphoreType.DMA,
    ],
)
def _sc_aggregate(hp_hbm, edges_hbm, out_hbm,
                  isrc_v, idst_v, rows_v, acc_sh, sem_i, sem_g, sem_s):
    cid, sid = _tile_ids()
    row0 = sid * ROWS_PER_TILE
    rchunks = [(q * CHUNK, min(CHUNK, ROWS_PER_TILE - q * CHUNK))
               for q in range((ROWS_PER_TILE + CHUNK - 1) // CHUNK)]
    tile_base = (cid * NS + sid) * (CHUNKS_PER_TILE * CHUNK)

    def eslice(half, i):
        return edges_hbm.at[pl.ds(half * E_PAD + tile_base + i * CHUNK, CHUNK)]

    def issue_idx(i, b):
        pltpu.async_copy(eslice(0, i), isrc_v.at[b], sem_i)
        pltpu.async_copy(eslice(1, i), idst_v.at[b], sem_i)

    def wait_idx(i, b):
        pltpu.make_async_copy(eslice(0, i), isrc_v.at[b], sem_i).wait()
        pltpu.make_async_copy(eslice(1, i), idst_v.at[b], sem_i).wait()

    def issue_gather(i, s):
        pltpu.async_copy(hp_hbm.at[isrc_v.at[s]], rows_v.at[s % 2], sem_g)

    def wait_gather(i, s):
        pltpu.make_async_copy(
            hp_hbm.at[isrc_v.at[s]], rows_v.at[s % 2], sem_g).wait()

    def issue_scatter(i, s):
        pltpu.async_copy(rows_v.at[s % 2], acc_sh.at[idst_v.at[s]],
                         sem_s, add=True)

    def wait_scatter(i, s):
        pltpu.make_async_copy(rows_v.at[s % 2], acc_sh.at[idst_v.at[s]],
                              sem_s).wait()

    def fill_zero(r, _):
        for j in range(D // 16):
            rows_v[0, r, pl.ds(j * 16, 16)] = jnp.zeros((16,), jnp.float32)
        return _

    lax.fori_loop(0, CHUNK, fill_zero, None)
    for r0, rn in rchunks:
        pltpu.sync_copy(rows_v.at[0, pl.ds(0, rn)],
                        acc_sh.at[pl.ds(row0 + r0, rn)])
    plsc.subcore_barrier()

    def steady(i, s):
        wait_idx(i + 1, (s + 1) % 4)
        wait_scatter(i - 1, (s + 3) % 4)
        issue_gather(i + 1, (s + 1) % 4)
        wait_gather(i, s)
        issue_scatter(i, s)
        issue_idx(i + 2, (s + 2) % 4)

    pltpu.sync_copy(eslice(0, 0), isrc_v.at[0])
    pltpu.sync_copy(eslice(1, 0), idst_v.at[0])
    issue_gather(0, 0)
    issue_idx(1, 1)
    wait_idx(1, 1)
    issue_gather(1, 1)
    wait_gather(0, 0)
    issue_scatter(0, 0)
    issue_idx(2, 2)

    def quad(g, _):
        for b in range(4):
            steady(4 * g + 1 + b, (1 + b) % 4)
        return _

    lax.fori_loop(0, (CHUNKS_PER_TILE - 4) // 4, quad, None)
    n = CHUNKS_PER_TILE
    steady(n - 3, (n - 3) % 4)
    i = n - 2
    wait_idx(i + 1, (i + 1) % 4)
    wait_scatter(i - 1, (i - 1) % 4)
    issue_gather(i + 1, (i + 1) % 4)
    wait_gather(i, i % 4)
    issue_scatter(i, i % 4)
    wait_gather(i + 1, (i + 1) % 4)
    issue_scatter(i + 1, (i + 1) % 4)
    wait_scatter(i, i % 4)
    wait_scatter(i + 1, (i + 1) % 4)
    plsc.subcore_barrier()

    pltpu.sync_copy(acc_sh.at[pl.ds(row0, ROWS_PER_TILE)],
                    out_hbm.at[pl.ds(cid * N_PAD + row0, ROWS_PER_TILE)])


def _dinv_block(da, db, g):
    deg = da[:, 0:1] + db[:, 0:1] + 1.0
    rows = g * 128 + lax.broadcasted_iota(jnp.int32, (128, 1), 0)
    return jnp.where(rows < N, lax.rsqrt(deg), 0.0)


def _tc_mm_body(x_ref, w_ref, o_ref):
    o_ref[...] = jnp.dot(x_ref[...], w_ref[...],
                         preferred_element_type=jnp.float32)


def _tc_scale_body(h_ref, da_ref, db_ref, o_ref):
    g = pl.program_id(0)
    dinv = _dinv_block(da_ref[...], db_ref[...], g)
    o_ref[...] = h_ref[...] * dinv


def _tc_mid_body(aa_ref, ab_ref, hp_ref, b_ref, w_ref, da_ref, db_ref, o_ref):
    g = pl.program_id(0)
    dinv = _dinv_block(da_ref[...], db_ref[...], g)
    z = jnp.maximum(
        dinv * (aa_ref[...] + ab_ref[...] + hp_ref[...]) + b_ref[...], 0.0)
    o_ref[...] = jnp.dot(z, w_ref[...],
                         preferred_element_type=jnp.float32) * dinv


def _tc_out_body(aa_ref, ab_ref, hp_ref, b_ref, da_ref, db_ref, o_ref):
    g = pl.program_id(0)
    dinv = _dinv_block(da_ref[...], db_ref[...], g)
    o_ref[...] = jnp.maximum(
        dinv * (aa_ref[...] + ab_ref[...] + hp_ref[...]) + b_ref[...], 0.0)


_row_spec = pl.BlockSpec((128, D), lambda g: (g, 0))
_w_spec = pl.BlockSpec((D, D), lambda g: (0, 0))
_b_spec = pl.BlockSpec((1, D), lambda g: (0, 0))
_da_spec = pl.BlockSpec((128, DW), lambda g: (g, 0))
_db_spec = pl.BlockSpec((128, DW), lambda g: (N_BLOCKS + g, 0))
_out_struct = jax.ShapeDtypeStruct((N_PAD, D), jnp.float32)


def _tc_mm(x_p, w):
    return pl.pallas_call(
        _tc_mm_body, grid=(N_BLOCKS,),
        in_specs=[_row_spec, _w_spec],
        out_specs=_row_spec, out_shape=_out_struct,
    )(x_p, w)


def _tc_scale(h, deg2):
    return pl.pallas_call(
        _tc_scale_body, grid=(N_BLOCKS,),
        in_specs=[_row_spec, _da_spec, _db_spec],
        out_specs=_row_spec, out_shape=_out_struct,
    )(h, deg2, deg2)


def _tc_mid(acc, hp, b, w, deg2):
    return pl.pallas_call(
        _tc_mid_body, grid=(N_BLOCKS,),
        in_specs=[_row_spec,
                  pl.BlockSpec((128, D), lambda g: (N_BLOCKS + g, 0)),
                  _row_spec, _b_spec, _w_spec, _da_spec, _db_spec],
        out_specs=_row_spec, out_shape=_out_struct,
    )(acc, acc, hp, b, w, deg2, deg2)


def _tc_out(acc, hp, b, deg2):
    return pl.pallas_call(
        _tc_out_body, grid=(N_BLOCKS,),
        in_specs=[_row_spec,
                  pl.BlockSpec((128, D), lambda g: (N_BLOCKS + g, 0)),
                  _row_spec, _b_spec, _da_spec, _db_spec],
        out_specs=_row_spec,
        out_shape=jax.ShapeDtypeStruct((N, D), jnp.float32),
    )(acc, acc, hp, b, deg2, deg2)


def kernel(x, edge_index, W1, b1, W2, b2):
    pad_e = E_PAD - E
    pad_idx = N + jnp.arange(pad_e, dtype=jnp.int32) % (N_PAD - N)
    edges_p = jnp.concatenate(
        [edge_index[0], pad_idx, edge_index[1], pad_idx])
    b1r = b1.reshape(1, D)
    b2r = b2.reshape(1, D)

    deg2 = _sc_degree(edges_p)
    h1u = _tc_mm(x, W1)
    h1p = _tc_scale(h1u, deg2)
    acc1 = _sc_aggregate(h1p, edges_p)
    h2p = _tc_mid(acc1, h1p, b1r, W2, deg2)
    acc2 = _sc_aggregate(h2p, edges_p)
    return _tc_out(acc2, h2p, b2r, deg2)
